# baseline (speedup 1.0000x reference)
import functools

import jax
import jax.numpy as jnp
from jax import lax
from jax.experimental import pallas as pl
from jax.experimental.pallas import tpu as pltpu
from jax.experimental.pallas import tpu_sc as plsc

N = 10000
E = 320000
D = 128
G = 64
NCLS = 10

NC, NS, L = 2, 16, 16
NW = NC * NS
K = 128
ETOT = E + N
CH = -(-ETOT // (NW * K))
EPAD = NW * CH * K
NP = 10240
SPAD = N + 16
RPS = NP // NS
NPOOL = 128
CHP = 3
NROWP = NW * CHP * K


def _edge_sc_body(h_hbm, ssrc_hbm, sdst_hbm, edges_hbm,
                  num_out, den_out,
                  ed_v, ssrc_v, sdst_v, w_v, rows_v, num_sh, den_sh,
                  sem_a, sem_b, sem_sn, sem_sd, sem_i):
    c = lax.axis_index("c")
    s = lax.axis_index("s")
    wid = c * NS + s

    pltpu.sync_copy(ssrc_hbm, ssrc_v)
    pltpu.sync_copy(sdst_hbm, sdst_v)

    zero16 = jnp.zeros((L,), jnp.float32)

    @plsc.parallel_loop(0, K, unroll=4)
    def _(r):
        for j in range(D // L):
            rows_v[r, pl.ds(j * L, L)] = zero16

    for j in range(K // L):
        w_v[pl.ds(j * L, L)] = zero16
    base = s * RPS
    for q in range(RPS // K):
        pltpu.sync_copy(rows_v, num_sh.at[pl.ds(base + q * K, K)])
        pltpu.sync_copy(w_v, den_sh.at[pl.ds(base + q * K, K)])
    plsc.subcore_barrier()

    hk = K // 2
    pltpu.sync_copy(edges_hbm.at[wid, 0], ed_v.at[0])

    def chunk(ch, _):
        slot = lax.rem(ch, 2)
        nslot = 1 - slot
        @pl.when(ch > 0)
        def _():
            pltpu.make_async_copy(rows_v, num_sh.at[ed_v.at[0, 1]], sem_sn).wait()
            pltpu.make_async_copy(w_v, den_sh.at[ed_v.at[0, 1]], sem_sd).wait()
            pltpu.make_async_copy(edges_hbm.at[wid, ch], ed_v.at[0], sem_i).wait()

        @pl.when(ch + 1 < CH)
        def _():
            pltpu.async_copy(edges_hbm.at[wid, ch + 1], ed_v.at[nslot], sem_i)

        ca = pltpu.async_copy(h_hbm.at[ed_v.at[slot, 0, pl.ds(0, hk)]],
                              rows_v.at[pl.ds(0, hk)], sem_a)
        cb = pltpu.async_copy(h_hbm.at[ed_v.at[slot, 0, pl.ds(hk, hk)]],
                              rows_v.at[pl.ds(hk, hk)], sem_b)
        @plsc.parallel_loop(0, K // L, unroll=2)
        def _(g):
            si = ed_v[slot, 0, pl.ds(g * L, L)]
            di = ed_v[slot, 1, pl.ds(g * L, L)]
            e = plsc.load_gather(ssrc_v, [si]) + plsc.load_gather(sdst_v, [di])
            e = jnp.where(e >= 0.0, e, 0.2 * e)
            w_v[pl.ds(g * L, L)] = jnp.exp(e)

        ca.wait()

        @plsc.parallel_loop(0, hk, unroll=8)
        def _(r):
            wv = plsc.load_gather(w_v, [jnp.zeros((L,), jnp.int32) + r])
            for j in range(D // L):
                rows_v[r, pl.ds(j * L, L)] = rows_v[r, pl.ds(j * L, L)] * wv

        cb.wait()

        @plsc.parallel_loop(hk, K, unroll=8)
        def _(r):
            wv = plsc.load_gather(w_v, [jnp.zeros((L,), jnp.int32) + r])
            for j in range(D // L):
                rows_v[r, pl.ds(j * L, L)] = rows_v[r, pl.ds(j * L, L)] * wv

        pltpu.async_copy(rows_v, num_sh.at[ed_v.at[slot, 1]], sem_sn, add=True)
        pltpu.async_copy(w_v, den_sh.at[ed_v.at[slot, 1]], sem_sd, add=True)
        return 0

    lax.fori_loop(0, CH, chunk, 0)
    pltpu.make_async_copy(rows_v, num_sh.at[ed_v.at[0, 1]], sem_sn).wait()
    pltpu.make_async_copy(w_v, den_sh.at[ed_v.at[0, 1]], sem_sd).wait()
    plsc.subcore_barrier()
    pltpu.sync_copy(num_sh.at[pl.ds(base, RPS)], num_out.at[c, pl.ds(base, RPS)])
    pltpu.sync_copy(den_sh.at[pl.ds(base, RPS)], den_out.at[c, pl.ds(base, RPS)])


def _pool_sc_body(hj_hbm, bidx_hbm, pool_out, bid_v, rows_v, pool_sh):
    c = lax.axis_index("c")
    s = lax.axis_index("s")
    wid = c * NS + s
    zero16 = jnp.zeros((L,), jnp.float32)

    def zrow(r, _):
        for j in range(D // L):
            rows_v[r, pl.ds(j * L, L)] = zero16
        return 0

    lax.fori_loop(0, K, zrow, 0)
    stripe = NPOOL // NS
    pltpu.sync_copy(rows_v.at[pl.ds(0, stripe)],
                    pool_sh.at[pl.ds(s * stripe, stripe)])
    pltpu.sync_copy(bidx_hbm.at[wid], bid_v)
    plsc.subcore_barrier()
    for ch in range(CHP):
        row0 = (wid * CHP + ch) * K
        pltpu.sync_copy(hj_hbm.at[pl.ds(row0, K)], rows_v)
        pltpu.sync_copy(rows_v, pool_sh.at[bid_v.at[ch]], add=True)
    plsc.subcore_barrier()
    pltpu.sync_copy(pool_sh.at[pl.ds(s * stripe, stripe)],
                    pool_out.at[c, pl.ds(s * stripe, stripe)])


BA = 2000
BC = 2000
BD = 1000


def _dense_body(h_ref, w_ref, a_ref, hh_ref, s_ref):
    hb = jnp.dot(h_ref[...], w_ref[...], preferred_element_type=jnp.float32)
    hh_ref[...] = hb
    ss = jnp.sum(hb * a_ref[0][None, :], axis=1)
    sd = jnp.sum(hb * a_ref[1][None, :], axis=1)
    s_ref[...] = jnp.stack([ss, sd])[:, :, None]


def _make_dense(interpret=False):
    return pl.pallas_call(
        _dense_body,
        grid=(N // BA,),
        in_specs=[
            pl.BlockSpec((BA, D), lambda i: (i, 0)),
            pl.BlockSpec((D, D), lambda i: (0, 0)),
            pl.BlockSpec((2, D), lambda i: (0, 0)),
        ],
        out_specs=[
            pl.BlockSpec((BA, D), lambda i: (i, 0)),
            pl.BlockSpec((2, BA, 1), lambda i: (0, i, 0)),
        ],
        out_shape=[
            jax.ShapeDtypeStruct((N, D), jnp.float32),
            jax.ShapeDtypeStruct((2, N, 1), jnp.float32),
        ],
        interpret=interpret,
    )


def _norm_body(num_ref, den_ref, bias_ref, g_ref, b_ref, pa_ref, y_ref, acc_ref):
    p = pl.program_id(0)
    i = pl.program_id(1)
    na = num_ref[0] + num_ref[1]
    d = den_ref[0, :, 0] + den_ref[1, :, 0]
    t = na / (d[:, None] + 1e-16) + bias_ref[0][None, :]

    @pl.when(jnp.logical_and(p == 0, i == 0))
    def _():
        acc_ref[...] = jnp.zeros_like(acc_ref)

    @pl.when(p == 0)
    def _():
        acc_ref[0, :] += jnp.sum(t, axis=0)
        acc_ref[1, :] += jnp.sum(t * t, axis=0)

    @pl.when(p == 1)
    def _():
        mu = acc_ref[0, :] * (1.0 / N)
        var = acc_ref[1, :] * (1.0 / N) - mu * mu
        z = ((t - mu[None, :]) * lax.rsqrt(var + 1e-5) * g_ref[0][None, :]
             + b_ref[0][None, :])
        a = pa_ref[0, 0]
        y_ref[...] = jnp.where(z >= 0.0, z, a * z)


def _make_norm(interpret=False):
    return pl.pallas_call(
        _norm_body,
        grid=(2, N // BC),
        in_specs=[
            pl.BlockSpec((2, BC, D), lambda p, i: (0, i, 0)),
            pl.BlockSpec((2, BC, 1), lambda p, i: (0, i, 0)),
            pl.BlockSpec((1, D), lambda p, i: (0, 0)),
            pl.BlockSpec((1, D), lambda p, i: (0, 0)),
            pl.BlockSpec((1, D), lambda p, i: (0, 0)),
            pl.BlockSpec((1, 1), lambda p, i: (0, 0)),
        ],
        out_specs=pl.BlockSpec((BC, D), lambda p, i: (i, 0)),
        out_shape=jax.ShapeDtypeStruct((N, D), jnp.float32),
        scratch_shapes=[pltpu.VMEM((2, D), jnp.float32)],
        interpret=interpret,
    )


def _norm_dense_body(num_ref, den_ref, bias_ref, g_ref, b_ref, pa_ref,
                     w_ref, a_ref, y_ref, hh_ref, s_ref, acc_ref):
    p = pl.program_id(0)
    i = pl.program_id(1)
    na = num_ref[0] + num_ref[1]
    d = den_ref[0, :, 0] + den_ref[1, :, 0]
    t = na / (d[:, None] + 1e-16) + bias_ref[0][None, :]

    @pl.when(jnp.logical_and(p == 0, i == 0))
    def _():
        acc_ref[...] = jnp.zeros_like(acc_ref)

    @pl.when(p == 0)
    def _():
        acc_ref[0, :] += jnp.sum(t, axis=0)
        acc_ref[1, :] += jnp.sum(t * t, axis=0)

    @pl.when(p == 1)
    def _():
        mu = acc_ref[0, :] * (1.0 / N)
        var = acc_ref[1, :] * (1.0 / N) - mu * mu
        z = ((t - mu[None, :]) * lax.rsqrt(var + 1e-5) * g_ref[0][None, :]
             + b_ref[0][None, :])
        a = pa_ref[0, 0]
        y = jnp.where(z >= 0.0, z, a * z)
        y_ref[...] = y
        hb = jnp.dot(y, w_ref[...], preferred_element_type=jnp.float32)
        hh_ref[...] = hb
        ss = jnp.sum(hb * a_ref[0][None, :], axis=1)
        sd = jnp.sum(hb * a_ref[1][None, :], axis=1)
        s_ref[...] = jnp.stack([ss, sd])[:, :, None]


def _make_norm_dense(interpret=False):
    return pl.pallas_call(
        _norm_dense_body,
        grid=(2, N // BC),
        in_specs=[
            pl.BlockSpec((2, BC, D), lambda p, i: (0, i, 0)),
            pl.BlockSpec((2, BC, 1), lambda p, i: (0, i, 0)),
            pl.BlockSpec((1, D), lambda p, i: (0, 0)),
            pl.BlockSpec((1, D), lambda p, i: (0, 0)),
            pl.BlockSpec((1, D), lambda p, i: (0, 0)),
            pl.BlockSpec((1, 1), lambda p, i: (0, 0)),
            pl.BlockSpec((D, D), lambda p, i: (0, 0)),
            pl.BlockSpec((2, D), lambda p, i: (0, 0)),
        ],
        out_specs=[
            pl.BlockSpec((BC, D), lambda p, i: (i, 0)),
            pl.BlockSpec((BC, D), lambda p, i: (i, 0)),
            pl.BlockSpec((2, BC, 1), lambda p, i: (0, i, 0)),
        ],
        out_shape=[
            jax.ShapeDtypeStruct((N, D), jnp.float32),
            jax.ShapeDtypeStruct((N, D), jnp.float32),
            jax.ShapeDtypeStruct((2, N, 1), jnp.float32),
        ],
        scratch_shapes=[pltpu.VMEM((2, D), jnp.float32)],
        interpret=interpret,
    )


def _hj_body(y1_ref, y2_ref, y3_ref, wjk_ref, bjk_ref, hj_ref):
    hcat = jnp.concatenate([y1_ref[...], y2_ref[...], y3_ref[...]], axis=1)
    hj_ref[...] = jnp.dot(hcat, wjk_ref[...],
                          preferred_element_type=jnp.float32) + bjk_ref[0][None, :]


def _make_hj(interpret=False):
    return pl.pallas_call(
        _hj_body,
        grid=(N // BD,),
        in_specs=[
            pl.BlockSpec((BD, D), lambda i: (i, 0)),
            pl.BlockSpec((BD, D), lambda i: (i, 0)),
            pl.BlockSpec((BD, D), lambda i: (i, 0)),
            pl.BlockSpec((3 * D, D), lambda i: (0, 0)),
            pl.BlockSpec((1, D), lambda i: (0, 0)),
        ],
        out_specs=pl.BlockSpec((BD, D), lambda i: (i, 0)),
        out_shape=jax.ShapeDtypeStruct((N, D), jnp.float32),
        interpret=interpret,
    )


def _cls_body(pool_ref, wc1_ref, bc1_ref, cg_ref, cb_ref, wc2_ref, bc2_ref,
              pa_ref, out_ref):
    pooled = pool_ref[0, :G] + pool_ref[1, :G]
    z = jnp.dot(pooled, wc1_ref[...],
                preferred_element_type=jnp.float32) + bc1_ref[0][None, :]
    mu = jnp.mean(z, axis=0, keepdims=True)
    zc = z - mu
    var = jnp.mean(zc * zc, axis=0, keepdims=True)
    z = zc * lax.rsqrt(var + 1e-5) * cg_ref[0][None, :] + cb_ref[0][None, :]
    a = pa_ref[0, 0]
    z = jnp.where(z >= 0.0, z, a * z)
    out_ref[...] = jnp.dot(z, wc2_ref[...],
                           preferred_element_type=jnp.float32) + bc2_ref[0][None, :]


def _make_cls(interpret=False):
    return pl.pallas_call(
        _cls_body,
        in_specs=[
            pl.BlockSpec((NC, NPOOL, D), lambda: (0, 0, 0)),
            pl.BlockSpec((D, D), lambda: (0, 0)),
            pl.BlockSpec((1, D), lambda: (0, 0)),
            pl.BlockSpec((1, D), lambda: (0, 0)),
            pl.BlockSpec((1, D), lambda: (0, 0)),
            pl.BlockSpec((D, NCLS), lambda: (0, 0)),
            pl.BlockSpec((1, NCLS), lambda: (0, 0)),
            pl.BlockSpec((1, 1), lambda: (0, 0)),
        ],
        out_specs=pl.BlockSpec((G, NCLS), lambda: (0, 0)),
        out_shape=jax.ShapeDtypeStruct((G, NCLS), jnp.float32),
        interpret=interpret,
    )


_dense_fwd = _make_dense()
_norm_fwd = _make_norm()
_norm_dense_fwd = _make_norm_dense()
_hj_fwd = _make_hj()
_cls_fwd = _make_cls()


_edge_sc_cache = []


def _edge_call(h, ssrc_p, sdst_p, edges_p):
    if not _edge_sc_cache:
        mesh = plsc.VectorSubcoreMesh(
            core_axis_name="c", subcore_axis_name="s",
            num_cores=NC, num_subcores=NS)
        _edge_sc_cache.append(functools.partial(
            pl.kernel,
            out_type=(
                jax.ShapeDtypeStruct((NC, NP, D), jnp.float32),
                jax.ShapeDtypeStruct((NC, NP), jnp.float32),
            ),
            mesh=mesh,
            compiler_params=pltpu.CompilerParams(needs_layout_passes=False),
            scratch_types=[
                pltpu.VMEM((2, 2, K), jnp.int32),
                pltpu.VMEM((SPAD,), jnp.float32),
                pltpu.VMEM((SPAD,), jnp.float32),
                pltpu.VMEM((K,), jnp.float32),
                pltpu.VMEM((K, D), jnp.float32),
                pltpu.VMEM_SHARED((NP, D), jnp.float32),
                pltpu.VMEM_SHARED((NP,), jnp.float32),
                pltpu.SemaphoreType.DMA,
                pltpu.SemaphoreType.DMA,
                pltpu.SemaphoreType.DMA,
                pltpu.SemaphoreType.DMA,
                pltpu.SemaphoreType.DMA,
            ],
        )(_edge_sc_body))
    return _edge_sc_cache[0](h, ssrc_p, sdst_p, edges_p)


_pool_sc_cache = []


def _pool_call(hj_pad, bidx):
    if not _pool_sc_cache:
        mesh = plsc.VectorSubcoreMesh(
            core_axis_name="c", subcore_axis_name="s",
            num_cores=NC, num_subcores=NS)
        _pool_sc_cache.append(functools.partial(
            pl.kernel,
            out_type=jax.ShapeDtypeStruct((NC, NPOOL, D), jnp.float32),
            mesh=mesh,
            compiler_params=pltpu.CompilerParams(needs_layout_passes=False),
            scratch_types=[
                pltpu.VMEM((CHP, K), jnp.int32),
                pltpu.VMEM((K, D), jnp.float32),
                pltpu.VMEM_SHARED((NPOOL, D), jnp.float32),
            ],
        )(_pool_sc_body))
    return _pool_sc_cache[0](hj_pad, bidx)


def kernel(x, edge_index, batch, Ws, a_src, a_dst, bias, bn_gamma, bn_beta,
           W_jk, b_jk, W_c1, b_c1, bn_cg, bn_cb, W_c2, b_c2, prelu_a):
    loop = jnp.arange(N, dtype=edge_index.dtype)
    src = jnp.concatenate([edge_index[0], loop])
    dst = jnp.concatenate([edge_index[1], loop])
    src_p = jnp.pad(src, (0, EPAD - ETOT)).reshape(NW, CH, K).astype(jnp.int32)
    dst_p = jnp.pad(dst, (0, EPAD - ETOT),
                    constant_values=N).reshape(NW, CH, K).astype(jnp.int32)
    edges_p = jnp.stack([src_p, dst_p], axis=2)
    pa = jnp.reshape(prelu_a.astype(jnp.float32), (1, 1))

    ys = []
    hh, s2 = _dense_fwd(x, Ws[0], jnp.stack([a_src[0], a_dst[0]]))
    for i in range(3):
        ssrc_p = jnp.pad(s2[0, :, 0], (0, SPAD - N))
        sdst_p = jnp.pad(s2[1, :, 0], (0, SPAD - N))
        num2, den2 = _edge_call(hh, ssrc_p, sdst_p, edges_p)
        if i < 2:
            y, hh, s2 = _norm_dense_fwd(
                num2[:, :N], den2[:, :N, None], bias[i].reshape(1, D),
                bn_gamma[i].reshape(1, D), bn_beta[i].reshape(1, D), pa,
                Ws[i + 1], jnp.stack([a_src[i + 1], a_dst[i + 1]]))
        else:
            y = _norm_fwd(num2[:, :N], den2[:, :N, None], bias[i].reshape(1, D),
                          bn_gamma[i].reshape(1, D), bn_beta[i].reshape(1, D),
                          pa)
        ys.append(y)

    hj = _hj_fwd(ys[0], ys[1], ys[2], W_jk, b_jk.reshape(1, D))
    hj_pad = jnp.pad(hj, ((0, NROWP - N), (0, 0)))
    bidx = jnp.pad(batch.astype(jnp.int32), (0, NROWP - N),
                   constant_values=G).reshape(NW, CHP, K)
    pool2 = _pool_call(hj_pad, bidx)
    return _cls_fwd(pool2, W_c1, b_c1.reshape(1, D), bn_cg.reshape(1, D),
                    bn_cb.reshape(1, D), W_c2, b_c2.reshape(1, NCLS), pa)


# speedup vs baseline: 1.0209x; 1.0209x over previous
import functools

import jax
import jax.numpy as jnp
from jax import lax
from jax.experimental import pallas as pl
from jax.experimental.pallas import tpu as pltpu
from jax.experimental.pallas import tpu_sc as plsc

N = 10000
E = 320000
D = 128
G = 64
NCLS = 10

NC, NS, L = 2, 16, 16
NW = NC * NS
K = 128
ETOT = E + N
CH = -(-ETOT // (NW * K))
EPAD = NW * CH * K
NP = 10240
SPAD = N + 16
RPS = NP // NS
NPOOL = 128
CHP = 3
NROWP = NW * CHP * K


def _edge_sc_body(h_hbm, ssrc_hbm, sdst_hbm, edges_hbm,
                  num_out, den_out,
                  ed_v, ssrc_v, sdst_v, w_v, rows_v, num_sh, den_sh,
                  sem_a, sem_b, sem_sn, sem_sd, sem_i):
    c = lax.axis_index("c")
    s = lax.axis_index("s")
    wid = c * NS + s

    pltpu.sync_copy(ssrc_hbm, ssrc_v)
    pltpu.sync_copy(sdst_hbm, sdst_v)

    zero16 = jnp.zeros((L,), jnp.float32)

    @plsc.parallel_loop(0, K, unroll=4)
    def _(r):
        for j in range(D // L):
            rows_v[r, pl.ds(j * L, L)] = zero16

    for j in range(K // L):
        w_v[pl.ds(j * L, L)] = zero16
    base = s * RPS
    for q in range(RPS // K):
        pltpu.sync_copy(rows_v, num_sh.at[pl.ds(base + q * K, K)])
        pltpu.sync_copy(w_v, den_sh.at[pl.ds(base + q * K, K)])
    plsc.subcore_barrier()

    hk = K // 2
    pltpu.sync_copy(edges_hbm.at[wid, 0], ed_v.at[0])

    def chunk(ch, _):
        slot = lax.rem(ch, 2)
        nslot = 1 - slot
        @pl.when(ch > 0)
        def _():
            pltpu.make_async_copy(rows_v, num_sh.at[ed_v.at[0, 1]], sem_sn).wait()
            pltpu.make_async_copy(w_v, den_sh.at[ed_v.at[0, 1]], sem_sd).wait()
            pltpu.make_async_copy(edges_hbm.at[wid, ch], ed_v.at[0], sem_i).wait()

        @pl.when(ch + 1 < CH)
        def _():
            pltpu.async_copy(edges_hbm.at[wid, ch + 1], ed_v.at[nslot], sem_i)

        ca = pltpu.async_copy(h_hbm.at[ed_v.at[slot, 0, pl.ds(0, hk)]],
                              rows_v.at[pl.ds(0, hk)], sem_a)
        cb = pltpu.async_copy(h_hbm.at[ed_v.at[slot, 0, pl.ds(hk, hk)]],
                              rows_v.at[pl.ds(hk, hk)], sem_b)
        @plsc.parallel_loop(0, K // L, unroll=2)
        def _(g):
            si = ed_v[slot, 0, pl.ds(g * L, L)]
            di = ed_v[slot, 1, pl.ds(g * L, L)]
            e = plsc.load_gather(ssrc_v, [si]) + plsc.load_gather(sdst_v, [di])
            e = jnp.where(e >= 0.0, e, 0.2 * e)
            w_v[pl.ds(g * L, L)] = jnp.exp(e)

        ca.wait()

        @plsc.parallel_loop(0, hk, unroll=8)
        def _(r):
            wv = plsc.load_gather(w_v, [jnp.zeros((L,), jnp.int32) + r])
            for j in range(D // L):
                rows_v[r, pl.ds(j * L, L)] = rows_v[r, pl.ds(j * L, L)] * wv

        cb.wait()

        @plsc.parallel_loop(hk, K, unroll=8)
        def _(r):
            wv = plsc.load_gather(w_v, [jnp.zeros((L,), jnp.int32) + r])
            for j in range(D // L):
                rows_v[r, pl.ds(j * L, L)] = rows_v[r, pl.ds(j * L, L)] * wv

        pltpu.async_copy(rows_v, num_sh.at[ed_v.at[slot, 1]], sem_sn, add=True)
        pltpu.async_copy(w_v, den_sh.at[ed_v.at[slot, 1]], sem_sd, add=True)
        return 0

    lax.fori_loop(0, CH, chunk, 0)
    pltpu.make_async_copy(rows_v, num_sh.at[ed_v.at[0, 1]], sem_sn).wait()
    pltpu.make_async_copy(w_v, den_sh.at[ed_v.at[0, 1]], sem_sd).wait()
    plsc.subcore_barrier()
    pltpu.sync_copy(num_sh.at[pl.ds(base, RPS)], num_out.at[c, pl.ds(base, RPS)])
    pltpu.sync_copy(den_sh.at[pl.ds(base, RPS)], den_out.at[c, pl.ds(base, RPS)])


def _pool_sc_body(hj_hbm, bidx_hbm, pool_out, bid_v, rows_v, pool_sh):
    c = lax.axis_index("c")
    s = lax.axis_index("s")
    wid = c * NS + s
    zero16 = jnp.zeros((L,), jnp.float32)

    def zrow(r, _):
        for j in range(D // L):
            rows_v[r, pl.ds(j * L, L)] = zero16
        return 0

    lax.fori_loop(0, K, zrow, 0)
    stripe = NPOOL // NS
    pltpu.sync_copy(rows_v.at[pl.ds(0, stripe)],
                    pool_sh.at[pl.ds(s * stripe, stripe)])
    pltpu.sync_copy(bidx_hbm.at[wid], bid_v)
    plsc.subcore_barrier()
    for ch in range(CHP):
        row0 = (wid * CHP + ch) * K
        pltpu.sync_copy(hj_hbm.at[pl.ds(row0, K)], rows_v)
        pltpu.sync_copy(rows_v, pool_sh.at[bid_v.at[ch]], add=True)
    plsc.subcore_barrier()
    pltpu.sync_copy(pool_sh.at[pl.ds(s * stripe, stripe)],
                    pool_out.at[c, pl.ds(s * stripe, stripe)])


BA = 2000
BC = 2000
BD = 1000


def _dense_body(h_ref, w_ref, a_ref, hh_ref, s_ref):
    hb = jnp.dot(h_ref[...], w_ref[...], preferred_element_type=jnp.float32)
    hh_ref[...] = hb
    ss = jnp.sum(hb * a_ref[0][None, :], axis=1)
    sd = jnp.sum(hb * a_ref[1][None, :], axis=1)
    s_ref[...] = jnp.stack([ss, sd])[:, :, None]


def _make_dense(interpret=False):
    return pl.pallas_call(
        _dense_body,
        grid=(N // BA,),
        in_specs=[
            pl.BlockSpec((BA, D), lambda i: (i, 0)),
            pl.BlockSpec((D, D), lambda i: (0, 0)),
            pl.BlockSpec((2, D), lambda i: (0, 0)),
        ],
        out_specs=[
            pl.BlockSpec((BA, D), lambda i: (i, 0)),
            pl.BlockSpec((2, BA, 1), lambda i: (0, i, 0)),
        ],
        out_shape=[
            jax.ShapeDtypeStruct((N, D), jnp.float32),
            jax.ShapeDtypeStruct((2, N, 1), jnp.float32),
        ],
        interpret=interpret,
    )


def _norm_body(num_ref, den_ref, bias_ref, g_ref, b_ref, pa_ref, y_ref, acc_ref):
    p = pl.program_id(0)
    i = pl.program_id(1)
    na = num_ref[0] + num_ref[1]
    d = den_ref[0, :, 0] + den_ref[1, :, 0]
    t = na / (d[:, None] + 1e-16) + bias_ref[0][None, :]

    @pl.when(jnp.logical_and(p == 0, i == 0))
    def _():
        acc_ref[...] = jnp.zeros_like(acc_ref)

    @pl.when(p == 0)
    def _():
        acc_ref[0, :] += jnp.sum(t, axis=0)
        acc_ref[1, :] += jnp.sum(t * t, axis=0)

    @pl.when(p == 1)
    def _():
        mu = acc_ref[0, :] * (1.0 / N)
        var = acc_ref[1, :] * (1.0 / N) - mu * mu
        z = ((t - mu[None, :]) * lax.rsqrt(var + 1e-5) * g_ref[0][None, :]
             + b_ref[0][None, :])
        a = pa_ref[0, 0]
        y_ref[...] = jnp.where(z >= 0.0, z, a * z)


def _make_norm(interpret=False):
    return pl.pallas_call(
        _norm_body,
        grid=(2, N // BC),
        in_specs=[
            pl.BlockSpec((2, BC, D), lambda p, i: (0, i, 0)),
            pl.BlockSpec((2, BC, 1), lambda p, i: (0, i, 0)),
            pl.BlockSpec((1, D), lambda p, i: (0, 0)),
            pl.BlockSpec((1, D), lambda p, i: (0, 0)),
            pl.BlockSpec((1, D), lambda p, i: (0, 0)),
            pl.BlockSpec((1, 1), lambda p, i: (0, 0)),
        ],
        out_specs=pl.BlockSpec((BC, D), lambda p, i: (i, 0)),
        out_shape=jax.ShapeDtypeStruct((N, D), jnp.float32),
        scratch_shapes=[pltpu.VMEM((2, D), jnp.float32)],
        interpret=interpret,
    )


def _hj_body(y1_ref, y2_ref, y3_ref, wjk_ref, bjk_ref, hj_ref):
    hcat = jnp.concatenate([y1_ref[...], y2_ref[...], y3_ref[...]], axis=1)
    hj_ref[...] = jnp.dot(hcat, wjk_ref[...],
                          preferred_element_type=jnp.float32) + bjk_ref[0][None, :]


def _make_hj(interpret=False):
    return pl.pallas_call(
        _hj_body,
        grid=(N // BD,),
        in_specs=[
            pl.BlockSpec((BD, D), lambda i: (i, 0)),
            pl.BlockSpec((BD, D), lambda i: (i, 0)),
            pl.BlockSpec((BD, D), lambda i: (i, 0)),
            pl.BlockSpec((3 * D, D), lambda i: (0, 0)),
            pl.BlockSpec((1, D), lambda i: (0, 0)),
        ],
        out_specs=pl.BlockSpec((BD, D), lambda i: (i, 0)),
        out_shape=jax.ShapeDtypeStruct((N, D), jnp.float32),
        interpret=interpret,
    )


def _cls_body(pool_ref, wc1_ref, bc1_ref, cg_ref, cb_ref, wc2_ref, bc2_ref,
              pa_ref, out_ref):
    pooled = pool_ref[0, :G] + pool_ref[1, :G]
    z = jnp.dot(pooled, wc1_ref[...],
                preferred_element_type=jnp.float32) + bc1_ref[0][None, :]
    mu = jnp.mean(z, axis=0, keepdims=True)
    zc = z - mu
    var = jnp.mean(zc * zc, axis=0, keepdims=True)
    z = zc * lax.rsqrt(var + 1e-5) * cg_ref[0][None, :] + cb_ref[0][None, :]
    a = pa_ref[0, 0]
    z = jnp.where(z >= 0.0, z, a * z)
    out_ref[...] = jnp.dot(z, wc2_ref[...],
                           preferred_element_type=jnp.float32) + bc2_ref[0][None, :]


def _make_cls(interpret=False):
    return pl.pallas_call(
        _cls_body,
        in_specs=[
            pl.BlockSpec((NC, NPOOL, D), lambda: (0, 0, 0)),
            pl.BlockSpec((D, D), lambda: (0, 0)),
            pl.BlockSpec((1, D), lambda: (0, 0)),
            pl.BlockSpec((1, D), lambda: (0, 0)),
            pl.BlockSpec((1, D), lambda: (0, 0)),
            pl.BlockSpec((D, NCLS), lambda: (0, 0)),
            pl.BlockSpec((1, NCLS), lambda: (0, 0)),
            pl.BlockSpec((1, 1), lambda: (0, 0)),
        ],
        out_specs=pl.BlockSpec((G, NCLS), lambda: (0, 0)),
        out_shape=jax.ShapeDtypeStruct((G, NCLS), jnp.float32),
        interpret=interpret,
    )


_dense_fwd = _make_dense()
_norm_fwd = _make_norm()
_hj_fwd = _make_hj()
_cls_fwd = _make_cls()


_edge_sc_cache = []


def _edge_call(h, ssrc_p, sdst_p, edges_p):
    if not _edge_sc_cache:
        mesh = plsc.VectorSubcoreMesh(
            core_axis_name="c", subcore_axis_name="s",
            num_cores=NC, num_subcores=NS)
        _edge_sc_cache.append(functools.partial(
            pl.kernel,
            out_type=(
                jax.ShapeDtypeStruct((NC, NP, D), jnp.float32),
                jax.ShapeDtypeStruct((NC, NP), jnp.float32),
            ),
            mesh=mesh,
            compiler_params=pltpu.CompilerParams(needs_layout_passes=False),
            scratch_types=[
                pltpu.VMEM((2, 2, K), jnp.int32),
                pltpu.VMEM((SPAD,), jnp.float32),
                pltpu.VMEM((SPAD,), jnp.float32),
                pltpu.VMEM((K,), jnp.float32),
                pltpu.VMEM((K, D), jnp.float32),
                pltpu.VMEM_SHARED((NP, D), jnp.float32),
                pltpu.VMEM_SHARED((NP,), jnp.float32),
                pltpu.SemaphoreType.DMA,
                pltpu.SemaphoreType.DMA,
                pltpu.SemaphoreType.DMA,
                pltpu.SemaphoreType.DMA,
                pltpu.SemaphoreType.DMA,
            ],
        )(_edge_sc_body))
    return _edge_sc_cache[0](h, ssrc_p, sdst_p, edges_p)


_pool_sc_cache = []


def _pool_call(hj_pad, bidx):
    if not _pool_sc_cache:
        mesh = plsc.VectorSubcoreMesh(
            core_axis_name="c", subcore_axis_name="s",
            num_cores=NC, num_subcores=NS)
        _pool_sc_cache.append(functools.partial(
            pl.kernel,
            out_type=jax.ShapeDtypeStruct((NC, NPOOL, D), jnp.float32),
            mesh=mesh,
            compiler_params=pltpu.CompilerParams(needs_layout_passes=False),
            scratch_types=[
                pltpu.VMEM((CHP, K), jnp.int32),
                pltpu.VMEM((K, D), jnp.float32),
                pltpu.VMEM_SHARED((NPOOL, D), jnp.float32),
            ],
        )(_pool_sc_body))
    return _pool_sc_cache[0](hj_pad, bidx)


def kernel(x, edge_index, batch, Ws, a_src, a_dst, bias, bn_gamma, bn_beta,
           W_jk, b_jk, W_c1, b_c1, bn_cg, bn_cb, W_c2, b_c2, prelu_a):
    loop = jnp.arange(N, dtype=edge_index.dtype)
    src = jnp.concatenate([edge_index[0], loop])
    dst = jnp.concatenate([edge_index[1], loop])
    src_p = jnp.pad(src, (0, EPAD - ETOT)).reshape(NW, CH, K).astype(jnp.int32)
    dst_p = jnp.pad(dst, (0, EPAD - ETOT),
                    constant_values=N).reshape(NW, CH, K).astype(jnp.int32)
    edges_p = jnp.stack([src_p, dst_p], axis=2)
    pa = jnp.reshape(prelu_a.astype(jnp.float32), (1, 1))

    h = x
    ys = []
    for i in range(3):
        a2 = jnp.stack([a_src[i], a_dst[i]])
        hh, s2 = _dense_fwd(h, Ws[i], a2)
        ssrc_p = jnp.pad(s2[0, :, 0], (0, SPAD - N))
        sdst_p = jnp.pad(s2[1, :, 0], (0, SPAD - N))
        num2, den2 = _edge_call(hh, ssrc_p, sdst_p, edges_p)
        h = _norm_fwd(num2[:, :N], den2[:, :N, None], bias[i].reshape(1, D),
                      bn_gamma[i].reshape(1, D), bn_beta[i].reshape(1, D), pa)
        ys.append(h)

    hj = _hj_fwd(ys[0], ys[1], ys[2], W_jk, b_jk.reshape(1, D))
    hj_pad = jnp.pad(hj, ((0, NROWP - N), (0, 0)))
    bidx = jnp.pad(batch.astype(jnp.int32), (0, NROWP - N),
                   constant_values=G).reshape(NW, CHP, K)
    pool2 = _pool_call(hj_pad, bidx)
    return _cls_fwd(pool2, W_c1, b_c1.reshape(1, D), bn_cg.reshape(1, D),
                    bn_cb.reshape(1, D), W_c2, b_c2.reshape(1, NCLS), pa)
